# baseline (speedup 1.0000x reference)
import functools

import jax
import jax.numpy as jnp
from jax.experimental import pallas as pl
from jax.experimental.pallas import tpu as pltpu

EMB = 128
L = 3


def _mm_kernel(x_ref, w_ref, b_ref, o_ref):
    o_ref[...] = (
        jnp.dot(x_ref[...], w_ref[...], preferred_element_type=jnp.float32)
        + b_ref[...]
    )


def _mm(x, w, b, blk):
    M, K = x.shape
    N = w.shape[1]
    return pl.pallas_call(
        _mm_kernel,
        grid=(M // blk,),
        in_specs=[
            pl.BlockSpec((blk, K), lambda i: (i, 0)),
            pl.BlockSpec((K, N), lambda i: (0, 0)),
            pl.BlockSpec((1, N), lambda i: (0, 0)),
        ],
        out_specs=pl.BlockSpec((blk, N), lambda i: (i, 0)),
        out_shape=jax.ShapeDtypeStruct((M, N), jnp.float32),
    )(x, w, b.reshape(1, N))


def _edge_up_kernel(e_ref, pe_ref, gs_ref, gr_ref, o_ref):
    o_ref[...] = e_ref[...] + jnp.maximum(
        pe_ref[...] + gs_ref[...] + gr_ref[...], 0.0
    )


def _edge_up(e, pe, gs, gr, blk=2000):
    M, N = e.shape
    spec = pl.BlockSpec((blk, N), lambda i: (i, 0))
    return pl.pallas_call(
        _edge_up_kernel,
        grid=(M // blk,),
        in_specs=[spec] * 4,
        out_specs=spec,
        out_shape=jax.ShapeDtypeStruct((M, N), jnp.float32),
    )(e, pe, gs, gr)


def _node_up_kernel(n_ref, s_ref, r_ref, w1_ref, w2_ref, w3_ref, c_ref, o_ref):
    acc = jnp.dot(n_ref[...], w1_ref[...], preferred_element_type=jnp.float32)
    acc += jnp.dot(s_ref[...], w2_ref[...], preferred_element_type=jnp.float32)
    acc += jnp.dot(r_ref[...], w3_ref[...], preferred_element_type=jnp.float32)
    acc += c_ref[...]
    o_ref[...] = n_ref[...] + jnp.maximum(acc, 0.0)


def _node_up(n, s, r, w1, w2, w3, c, blk=2000):
    M, N = n.shape
    dspec = pl.BlockSpec((blk, N), lambda i: (i, 0))
    wspec = pl.BlockSpec((N, N), lambda i: (0, 0))
    return pl.pallas_call(
        _node_up_kernel,
        grid=(M // blk,),
        in_specs=[dspec, dspec, dspec, wspec, wspec, wspec,
                  pl.BlockSpec((1, N), lambda i: (0, 0))],
        out_specs=dspec,
        out_shape=jax.ShapeDtypeStruct((M, N), jnp.float32),
    )(n, s, r, w1, w2, w3, c)


def _seg_max(data, ids, num):
    out = jax.ops.segment_max(data, ids, num_segments=num)
    return jnp.where(jnp.isfinite(out), out, 0.0)


def kernel(nodes, edges, globals_, senders, receivers, We_emb, be_emb,
           Wn_emb, bn_emb, Wg_emb, bg_emb, W_edge, b_edge, W_node, b_node,
           W_glob, b_glob, W_dec, b_dec):
    N = nodes.shape[0]
    E = edges.shape[0]

    e = _mm(edges, We_emb, be_emb, blk=2000)
    n = _mm(nodes, Wn_emb, bn_emb, blk=2000)
    g = globals_ @ Wg_emb + bg_emb

    We1, We2, We3, We4 = (W_edge[:, i * EMB:(i + 1) * EMB, :] for i in range(4))
    Wn1, Wn2, Wn3, Wn4 = (W_node[:, i * EMB:(i + 1) * EMB, :] for i in range(4))

    for l in range(L):
        ec = g @ We4[l] + b_edge[l]
        pe = _mm(e, We1[l], ec[0], blk=2000)
        ts = _mm(n, We2[l], jnp.zeros((EMB,), jnp.float32), blk=2000)
        tr = _mm(n, We3[l], jnp.zeros((EMB,), jnp.float32), blk=2000)
        gs = ts[senders]
        gr = tr[receivers]
        e = _edge_up(e, pe, gs, gr, blk=2000)

        sent = _seg_max(e, senders, N)
        recv = _seg_max(e, receivers, N)
        nc = g @ Wn4[l] + b_node[l]
        n = _node_up(n, sent, recv, Wn1[l], Wn2[l], Wn3[l], nc, blk=2000)

        agg_n = jnp.max(n, axis=0, keepdims=True)
        agg_e = jnp.max(e, axis=0, keepdims=True)
        Wg1 = W_glob[l, :EMB]
        Wg2 = W_glob[l, EMB:2 * EMB]
        Wg3 = W_glob[l, 2 * EMB:]
        g = g + jax.nn.relu(agg_n @ Wg1 + agg_e @ Wg2 + g @ Wg3 + b_glob[l])

    e_dec = _mm(e, W_dec, b_dec, blk=2000)
    return n, e_dec, g


# speedup vs baseline: 1.6482x; 1.6482x over previous
import functools

import jax
import jax.numpy as jnp
from jax import lax
from jax.experimental import pallas as pl
from jax.experimental.pallas import tpu as pltpu
from jax.experimental.pallas import tpu_sc as plsc

EMB = 128
E_ = 160000
N_ = 10000
NC = 2
NS = 16
NW = NC * NS
EPW = E_ // NW
SCH = 128
NCHE = EPW // SCH + 1
NPW = 320
NPAD = NW * NPW
EPAD = E_ + 160
RSP = 10304


def _mm_kernel(x_ref, w_ref, b_ref, o_ref):
    o_ref[...] = (
        jnp.dot(x_ref[...], w_ref[...], preferred_element_type=jnp.float32)
        + b_ref[...]
    )


def _mm(x, w, b, blk):
    M, K = x.shape
    N = w.shape[1]
    return pl.pallas_call(
        _mm_kernel,
        grid=(M // blk,),
        in_specs=[
            pl.BlockSpec((blk, K), lambda i: (i, 0)),
            pl.BlockSpec((K, N), lambda i: (0, 0)),
            pl.BlockSpec((1, N), lambda i: (0, 0)),
        ],
        out_specs=pl.BlockSpec((blk, N), lambda i: (i, 0)),
        out_shape=jax.ShapeDtypeStruct((M, N), jnp.float32),
    )(x, w, b.reshape(1, N))


def _mm_nb_kernel(x_ref, w_ref, o_ref):
    o_ref[...] = jnp.dot(x_ref[...], w_ref[...],
                         preferred_element_type=jnp.float32)


def _mm_nb(x, w, blk):
    M, K = x.shape
    N = w.shape[1]
    return pl.pallas_call(
        _mm_nb_kernel,
        grid=(M // blk,),
        in_specs=[
            pl.BlockSpec((blk, K), lambda i: (i, 0)),
            pl.BlockSpec((K, N), lambda i: (0, 0)),
        ],
        out_specs=pl.BlockSpec((blk, N), lambda i: (i, 0)),
        out_shape=jax.ShapeDtypeStruct((M, N), jnp.float32),
    )(x, w)


def _mm2_kernel(x_ref, w1_ref, w2_ref, o1_ref, o2_ref):
    o1_ref[...] = jnp.dot(x_ref[...], w1_ref[...],
                          preferred_element_type=jnp.float32)
    o2_ref[...] = jnp.dot(x_ref[...], w2_ref[...],
                          preferred_element_type=jnp.float32)


def _mm2(x, w1, w2, blk):
    M, K = x.shape
    N = w1.shape[1]
    dspec = pl.BlockSpec((blk, N), lambda i: (i, 0))
    return pl.pallas_call(
        _mm2_kernel,
        grid=(M // blk,),
        in_specs=[
            pl.BlockSpec((blk, K), lambda i: (i, 0)),
            pl.BlockSpec((K, N), lambda i: (0, 0)),
            pl.BlockSpec((K, N), lambda i: (0, 0)),
        ],
        out_specs=[dspec, dspec],
        out_shape=[jax.ShapeDtypeStruct((M, N), jnp.float32)] * 2,
    )(x, w1, w2)


def _node_up_kernel(n_ref, s_ref, r_ref, w1_ref, w2_ref, w3_ref, c_ref,
                    o_ref, aggn_ref):
    acc = jnp.dot(n_ref[...], w1_ref[...], preferred_element_type=jnp.float32)
    acc += jnp.dot(s_ref[...], w2_ref[...], preferred_element_type=jnp.float32)
    acc += jnp.dot(r_ref[...], w3_ref[...], preferred_element_type=jnp.float32)
    acc += c_ref[...]
    nn = n_ref[...] + jnp.maximum(acc, 0.0)
    o_ref[...] = nn
    m = jnp.max(nn, axis=0, keepdims=True)
    i = pl.program_id(0)

    @pl.when(i == 0)
    def _():
        aggn_ref[...] = m

    @pl.when(i > 0)
    def _():
        aggn_ref[...] = jnp.maximum(aggn_ref[...], m)


def _node_up(n, s, r, w1, w2, w3, c, blk=2000):
    M = n.shape[0]
    dspec = pl.BlockSpec((blk, EMB), lambda i: (i, 0))
    wspec = pl.BlockSpec((EMB, EMB), lambda i: (0, 0))
    cspec = pl.BlockSpec((1, EMB), lambda i: (0, 0))
    return pl.pallas_call(
        _node_up_kernel,
        grid=(M // blk,),
        in_specs=[dspec, dspec, dspec, wspec, wspec, wspec, cspec],
        out_specs=[dspec, cspec],
        out_shape=[jax.ShapeDtypeStruct((M, EMB), jnp.float32),
                   jax.ShapeDtypeStruct((1, EMB), jnp.float32)],
    )(n, s, r, w1, w2, w3, c)


_MESH = plsc.VectorSubcoreMesh(core_axis_name="c", subcore_axis_name="s")


def _sc_edge_body(e_hbm, pe_hbm, ts_hbm, tr_hbm, snd_hbm, rcv_hbm, ec_hbm,
                  eout_hbm, agg_hbm,
                  ebuf, pbuf, tsbuf, trbuf, sidx, ridx, ecv, aggv,
                  sem1, sem2):
    wid = lax.axis_index("s") * NC + lax.axis_index("c")
    base = wid * EPW
    pltpu.sync_copy(ec_hbm, ecv)
    neg = jnp.full((16,), -3.0e38, jnp.float32)
    ecks = [ecv[pl.ds(k * 16, 16)] for k in range(8)]

    def chunk(c, aggs):
        off = base + pl.multiple_of(
            jnp.where(c == NCHE - 1, EPW - SCH, c * SCH), 8)
        c1 = pltpu.async_copy(snd_hbm.at[pl.ds(off, SCH)], sidx, sem1)
        c2 = pltpu.async_copy(rcv_hbm.at[pl.ds(off, SCH)], ridx, sem1)
        c3 = pltpu.async_copy(e_hbm.at[pl.ds(off, SCH)], ebuf, sem1)
        c4 = pltpu.async_copy(pe_hbm.at[pl.ds(off, SCH)], pbuf, sem1)
        c1.wait()
        c2.wait()
        c5 = pltpu.async_copy(ts_hbm.at[sidx], tsbuf, sem2)
        c6 = pltpu.async_copy(tr_hbm.at[ridx], trbuf, sem2)
        c3.wait()
        c4.wait()
        c5.wait()
        c6.wait()

        def row(t, ags):
            eb, pb, tb, rb = (ebuf.at[t], pbuf.at[t], tsbuf.at[t],
                              trbuf.at[t])
            out = []
            for k in range(8):
                sl = pl.ds(k * 16, 16)
                nv = eb[sl] + jnp.maximum(pb[sl] + tb[sl] + rb[sl] + ecks[k],
                                          0.0)
                eb[sl] = nv
                out.append(jnp.maximum(ags[k], nv))
            return tuple(out)

        new_aggs = lax.fori_loop(0, SCH, row, aggs)
        pltpu.sync_copy(ebuf, eout_hbm.at[pl.ds(off, SCH)])
        return new_aggs

    aggs = lax.fori_loop(0, NCHE, chunk, tuple(neg for _ in range(8)))
    for k in range(8):
        aggv[pl.ds(k * 16, 16)] = aggs[k]
    pltpu.sync_copy(aggv, agg_hbm.at[wid])


_SC_EDGE = pl.kernel(
    _sc_edge_body,
    out_type=[jax.ShapeDtypeStruct((E_, EMB), jnp.float32),
              jax.ShapeDtypeStruct((NW, EMB), jnp.float32)],
    mesh=_MESH,
    scratch_types=[pltpu.VMEM((SCH, EMB), jnp.float32),
                   pltpu.VMEM((SCH, EMB), jnp.float32),
                   pltpu.VMEM((SCH, EMB), jnp.float32),
                   pltpu.VMEM((SCH, EMB), jnp.float32),
                   pltpu.VMEM((SCH,), jnp.int32),
                   pltpu.VMEM((SCH,), jnp.int32),
                   pltpu.VMEM((EMB,), jnp.float32),
                   pltpu.VMEM((EMB,), jnp.float32),
                   pltpu.SemaphoreType.DMA,
                   pltpu.SemaphoreType.DMA],
)


def _sc_segmax_body(enew_hbm, perm_s_hbm, ids_s_hbm, rs_s_hbm,
                    perm_r_hbm, ids_r_hbm, rs_r_hbm,
                    sent_hbm, recv_hbm,
                    outb, rows, pv, iv, rsv, sem):
    wid = lax.axis_index("s") * NC + lax.axis_index("c")
    nstart = wid * NPW
    iota = lax.iota(jnp.int32, 16)
    zero16 = jnp.zeros((16,), jnp.float32)

    def read_rs(rs_h, pos):
        a = pl.multiple_of((pos // 8) * 8, 8)
        pltpu.sync_copy(rs_h.at[pl.ds(a, 24)], rsv)
        return rsv[pl.ds(pos - a, 16)][0]

    for perm_h, ids_h, rs_h, out_h in (
            (perm_s_hbm, ids_s_hbm, rs_s_hbm, sent_hbm),
            (perm_r_hbm, ids_r_hbm, rs_r_hbm, recv_hbm)):

        def zr(t, _):
            ob = outb.at[t]
            for k in range(8):
                ob[pl.ds(k * 16, 16)] = zero16
            return 0

        lax.fori_loop(0, NPW, zr, 0)

        est = read_rs(rs_h, nstart)
        een = read_rs(rs_h, nstart + NPW)
        est_a = pl.multiple_of((est // 8) * 8, 8)
        nch = (een - est_a + SCH - 1) // SCH

        def chunk(c, carry, perm_h=perm_h, ids_h=ids_h):
            off = pl.multiple_of(est_a + c * SCH, 8)
            pltpu.sync_copy(ids_h.at[pl.ds(off, SCH)], iv.at[pl.ds(0, SCH)])
            pltpu.sync_copy(perm_h.at[pl.ds(off, SCH)], pv)
            pltpu.async_copy(enew_hbm.at[pv], rows, sem).wait()

            def row(j, cr):
                pid, plid, ac = cr
                idd = iv[pl.ds(j, 16)][0]
                lid = idd - nstart
                is_new = idd != pid

                @pl.when(is_new & (plid >= 0) & (plid < NPW))
                def _():
                    ob = outb.at[plid]
                    for k in range(8):
                        ob[pl.ds(k * 16, 16)] = ac[k]

                rj = rows.at[j]
                nac = []
                for k in range(8):
                    rv = rj[pl.ds(k * 16, 16)]
                    nac.append(jnp.where(is_new, rv,
                                         jnp.maximum(ac[k], rv)))
                return (idd, lid, tuple(nac))

            return lax.fori_loop(0, SCH, row, carry)

        init = (jnp.int32(-(1 << 30)), jnp.int32(-1),
                tuple(zero16 for _ in range(8)))
        pid, plid, ac = lax.fori_loop(0, nch, chunk, init)

        @pl.when((plid >= 0) & (plid < NPW))
        def _():
            ob = outb.at[plid]
            for k in range(8):
                ob[pl.ds(k * 16, 16)] = ac[k]

        pltpu.sync_copy(outb, out_h.at[pl.ds(nstart, NPW)])


_SC_SEGMAX = pl.kernel(
    _sc_segmax_body,
    out_type=[jax.ShapeDtypeStruct((NPAD, EMB), jnp.float32),
              jax.ShapeDtypeStruct((NPAD, EMB), jnp.float32)],
    mesh=_MESH,
    scratch_types=[pltpu.VMEM((NPW, EMB), jnp.float32),
                   pltpu.VMEM((SCH, EMB), jnp.float32),
                   pltpu.VMEM((SCH,), jnp.int32),
                   pltpu.VMEM((SCH + 16,), jnp.int32),
                   pltpu.VMEM((24,), jnp.int32),
                   pltpu.SemaphoreType.DMA],
)


def _sorted_index_setup(ids):
    order = jnp.argsort(ids).astype(jnp.int32)
    ids_sorted = ids[order].astype(jnp.int32)
    rs = jnp.searchsorted(ids_sorted, jnp.arange(RSP, dtype=jnp.int32),
                          side="left").astype(jnp.int32)
    ids_pad = jnp.concatenate(
        [ids_sorted, jnp.full((EPAD - E_,), 1 << 22, jnp.int32)])
    perm_pad = jnp.concatenate(
        [order, jnp.zeros((EPAD - E_,), jnp.int32)])
    return perm_pad, ids_pad, rs


def kernel(nodes, edges, globals_, senders, receivers, We_emb, be_emb,
           Wn_emb, bn_emb, Wg_emb, bg_emb, W_edge, b_edge, W_node, b_node,
           W_glob, b_glob, W_dec, b_dec):
    e = _mm(edges, We_emb, be_emb, blk=2000)
    n = _mm(nodes, Wn_emb, bn_emb, blk=2000)
    g = globals_ @ Wg_emb + bg_emb

    perm_s, ids_s, rs_s = _sorted_index_setup(senders)
    perm_r, ids_r, rs_r = _sorted_index_setup(receivers)

    We1, We2, We3, We4 = (W_edge[:, i * EMB:(i + 1) * EMB, :] for i in range(4))
    Wn1, Wn2, Wn3, Wn4 = (W_node[:, i * EMB:(i + 1) * EMB, :] for i in range(4))
    Wg1 = W_glob[:, :EMB]
    Wg2 = W_glob[:, EMB:2 * EMB]
    Wg3 = W_glob[:, 2 * EMB:]

    for l in range(3):
        ec = (g @ We4[l] + b_edge[l]).reshape(EMB)
        pe = _mm_nb(e, We1[l], blk=2000)
        ts, tr = _mm2(n, We2[l], We3[l], blk=2000)
        e, agg32 = _SC_EDGE(e, pe, ts, tr, senders, receivers, ec)

        sent, recv = _SC_SEGMAX(e, perm_s, ids_s, rs_s, perm_r, ids_r, rs_r)

        nc = (g @ Wn4[l] + b_node[l]).reshape(1, EMB)
        n, aggn = _node_up(n, sent, recv, Wn1[l], Wn2[l], Wn3[l], nc)

        agge = jnp.max(agg32, axis=0, keepdims=True)
        g = g + jax.nn.relu(aggn @ Wg1[l] + agge @ Wg2[l] + g @ Wg3[l]
                            + b_glob[l])

    e_dec = _mm(e, W_dec, b_dec, blk=2000)
    return n, e_dec, g


# speedup vs baseline: 1.7204x; 1.0438x over previous
import functools

import jax
import jax.numpy as jnp
from jax import lax
from jax.experimental import pallas as pl
from jax.experimental.pallas import tpu as pltpu
from jax.experimental.pallas import tpu_sc as plsc

EMB = 128
E_ = 160000
N_ = 10000
NC = 2
NS = 16
NW = NC * NS
EPW = E_ // NW
SCH = 128
NCHE = EPW // SCH + 1
NPW = 320
NPAD = NW * NPW
EPAD = E_ + 160
RSP = 10304


def _mm_kernel(x_ref, w_ref, b_ref, o_ref):
    o_ref[...] = (
        jnp.dot(x_ref[...], w_ref[...], preferred_element_type=jnp.float32)
        + b_ref[...]
    )


def _mm(x, w, b, blk):
    M, K = x.shape
    N = w.shape[1]
    return pl.pallas_call(
        _mm_kernel,
        grid=(M // blk,),
        in_specs=[
            pl.BlockSpec((blk, K), lambda i: (i, 0)),
            pl.BlockSpec((K, N), lambda i: (0, 0)),
            pl.BlockSpec((1, N), lambda i: (0, 0)),
        ],
        out_specs=pl.BlockSpec((blk, N), lambda i: (i, 0)),
        out_shape=jax.ShapeDtypeStruct((M, N), jnp.float32),
    )(x, w, b.reshape(1, N))


def _mm_nb_kernel(x_ref, w_ref, o_ref):
    o_ref[...] = jnp.dot(x_ref[...], w_ref[...],
                         preferred_element_type=jnp.float32)


def _mm_nb(x, w, blk):
    M, K = x.shape
    N = w.shape[1]
    return pl.pallas_call(
        _mm_nb_kernel,
        grid=(M // blk,),
        in_specs=[
            pl.BlockSpec((blk, K), lambda i: (i, 0)),
            pl.BlockSpec((K, N), lambda i: (0, 0)),
        ],
        out_specs=pl.BlockSpec((blk, N), lambda i: (i, 0)),
        out_shape=jax.ShapeDtypeStruct((M, N), jnp.float32),
    )(x, w)


def _mm2_kernel(x_ref, w1_ref, w2_ref, o1_ref, o2_ref):
    o1_ref[...] = jnp.dot(x_ref[...], w1_ref[...],
                          preferred_element_type=jnp.float32)
    o2_ref[...] = jnp.dot(x_ref[...], w2_ref[...],
                          preferred_element_type=jnp.float32)


def _mm2(x, w1, w2, blk):
    M, K = x.shape
    N = w1.shape[1]
    dspec = pl.BlockSpec((blk, N), lambda i: (i, 0))
    return pl.pallas_call(
        _mm2_kernel,
        grid=(M // blk,),
        in_specs=[
            pl.BlockSpec((blk, K), lambda i: (i, 0)),
            pl.BlockSpec((K, N), lambda i: (0, 0)),
            pl.BlockSpec((K, N), lambda i: (0, 0)),
        ],
        out_specs=[dspec, dspec],
        out_shape=[jax.ShapeDtypeStruct((M, N), jnp.float32)] * 2,
    )(x, w1, w2)


def _node_up_kernel(n_ref, s_ref, r_ref, w1_ref, w2_ref, w3_ref, c_ref,
                    o_ref, aggn_ref):
    acc = jnp.dot(n_ref[...], w1_ref[...], preferred_element_type=jnp.float32)
    acc += jnp.dot(s_ref[...], w2_ref[...], preferred_element_type=jnp.float32)
    acc += jnp.dot(r_ref[...], w3_ref[...], preferred_element_type=jnp.float32)
    acc += c_ref[...]
    nn = n_ref[...] + jnp.maximum(acc, 0.0)
    o_ref[...] = nn
    m = jnp.max(nn, axis=0, keepdims=True)
    i = pl.program_id(0)

    @pl.when(i == 0)
    def _():
        aggn_ref[...] = m

    @pl.when(i > 0)
    def _():
        aggn_ref[...] = jnp.maximum(aggn_ref[...], m)


def _node_up(n, s, r, w1, w2, w3, c, blk=2000):
    M = n.shape[0]
    dspec = pl.BlockSpec((blk, EMB), lambda i: (i, 0))
    wspec = pl.BlockSpec((EMB, EMB), lambda i: (0, 0))
    cspec = pl.BlockSpec((1, EMB), lambda i: (0, 0))
    return pl.pallas_call(
        _node_up_kernel,
        grid=(M // blk,),
        in_specs=[dspec, dspec, dspec, wspec, wspec, wspec, cspec],
        out_specs=[dspec, cspec],
        out_shape=[jax.ShapeDtypeStruct((M, EMB), jnp.float32),
                   jax.ShapeDtypeStruct((1, EMB), jnp.float32)],
    )(n, s, r, w1, w2, w3, c)


_MESH = plsc.VectorSubcoreMesh(core_axis_name="c", subcore_axis_name="s")


def _sc_gather_body(ts_hbm, tr_hbm, snd_hbm, rcv_hbm,
                    gs_hbm, gr_hbm,
                    tsbuf, trbuf, sidx, ridx, sem1, sem2):
    wid = lax.axis_index("s") * NC + lax.axis_index("c")
    base = wid * EPW

    def chunk(c, _):
        off = base + pl.multiple_of(
            jnp.where(c == NCHE - 1, EPW - SCH, c * SCH), 8)
        c1 = pltpu.async_copy(snd_hbm.at[pl.ds(off, SCH)], sidx, sem1)
        c2 = pltpu.async_copy(rcv_hbm.at[pl.ds(off, SCH)], ridx, sem1)
        c1.wait()
        c2.wait()
        c5 = pltpu.async_copy(ts_hbm.at[sidx], tsbuf, sem2)
        c6 = pltpu.async_copy(tr_hbm.at[ridx], trbuf, sem2)
        c5.wait()
        c6.wait()
        pltpu.sync_copy(tsbuf, gs_hbm.at[pl.ds(off, SCH)])
        pltpu.sync_copy(trbuf, gr_hbm.at[pl.ds(off, SCH)])
        return 0

    lax.fori_loop(0, NCHE, chunk, 0)


_SC_GATHER = pl.kernel(
    _sc_gather_body,
    out_type=[jax.ShapeDtypeStruct((E_, EMB), jnp.float32),
              jax.ShapeDtypeStruct((E_, EMB), jnp.float32)],
    mesh=_MESH,
    scratch_types=[pltpu.VMEM((SCH, EMB), jnp.float32),
                   pltpu.VMEM((SCH, EMB), jnp.float32),
                   pltpu.VMEM((SCH,), jnp.int32),
                   pltpu.VMEM((SCH,), jnp.int32),
                   pltpu.SemaphoreType.DMA,
                   pltpu.SemaphoreType.DMA],
)


def _edge_up_kernel(e_ref, gs_ref, gr_ref, w_ref, c_ref, o_ref, agg_ref):
    acc = jnp.dot(e_ref[...], w_ref[...], preferred_element_type=jnp.float32)
    acc += gs_ref[...] + gr_ref[...] + c_ref[...]
    nn = e_ref[...] + jnp.maximum(acc, 0.0)
    o_ref[...] = nn
    m = jnp.max(nn, axis=0, keepdims=True)
    i = pl.program_id(0)

    @pl.when(i == 0)
    def _():
        agg_ref[...] = m

    @pl.when(i > 0)
    def _():
        agg_ref[...] = jnp.maximum(agg_ref[...], m)


def _edge_up(e, gs, gr, w, c, blk=2000):
    M = e.shape[0]
    dspec = pl.BlockSpec((blk, EMB), lambda i: (i, 0))
    wspec = pl.BlockSpec((EMB, EMB), lambda i: (0, 0))
    cspec = pl.BlockSpec((1, EMB), lambda i: (0, 0))
    return pl.pallas_call(
        _edge_up_kernel,
        grid=(M // blk,),
        in_specs=[dspec, dspec, dspec, wspec, cspec],
        out_specs=[dspec, cspec],
        out_shape=[jax.ShapeDtypeStruct((M, EMB), jnp.float32),
                   jax.ShapeDtypeStruct((1, EMB), jnp.float32)],
    )(e, gs, gr, w, c)


def _sc_segmax_body(enew_hbm, perm_s_hbm, ids_s_hbm, rs_s_hbm,
                    perm_r_hbm, ids_r_hbm, rs_r_hbm,
                    sent_hbm, recv_hbm,
                    outb, rows, pv, iv, rsv, sem):
    wid = lax.axis_index("s") * NC + lax.axis_index("c")
    nstart = wid * NPW
    iota = lax.iota(jnp.int32, 16)
    zero16 = jnp.zeros((16,), jnp.float32)

    def read_rs(rs_h, pos):
        a = pl.multiple_of((pos // 8) * 8, 8)
        pltpu.sync_copy(rs_h.at[pl.ds(a, 24)], rsv)
        return rsv[pl.ds(pos - a, 16)][0]

    for perm_h, ids_h, rs_h, out_h in (
            (perm_s_hbm, ids_s_hbm, rs_s_hbm, sent_hbm),
            (perm_r_hbm, ids_r_hbm, rs_r_hbm, recv_hbm)):

        def zr(t, _):
            ob = outb.at[t]
            for k in range(8):
                ob[pl.ds(k * 16, 16)] = zero16
            return 0

        lax.fori_loop(0, NPW, zr, 0)

        est = read_rs(rs_h, nstart)
        een = read_rs(rs_h, nstart + NPW)
        est_a = pl.multiple_of((est // 8) * 8, 8)
        nch = (een - est_a + SCH - 1) // SCH

        def chunk(c, carry, perm_h=perm_h, ids_h=ids_h):
            off = pl.multiple_of(est_a + c * SCH, 8)
            pltpu.sync_copy(ids_h.at[pl.ds(off, SCH)], iv.at[pl.ds(0, SCH)])
            pltpu.sync_copy(perm_h.at[pl.ds(off, SCH)], pv)
            pltpu.async_copy(enew_hbm.at[pv], rows, sem).wait()

            def row(j, cr):
                pid, plid, ac = cr
                idd = iv[pl.ds(j, 16)][0]
                lid = idd - nstart
                is_new = idd != pid

                @pl.when(is_new & (plid >= 0) & (plid < NPW))
                def _():
                    ob = outb.at[plid]
                    for k in range(8):
                        ob[pl.ds(k * 16, 16)] = ac[k]

                rj = rows.at[j]
                nac = []
                for k in range(8):
                    rv = rj[pl.ds(k * 16, 16)]
                    nac.append(jnp.where(is_new, rv,
                                         jnp.maximum(ac[k], rv)))
                return (idd, lid, tuple(nac))

            return lax.fori_loop(0, SCH, row, carry)

        init = (jnp.int32(-(1 << 30)), jnp.int32(-1),
                tuple(zero16 for _ in range(8)))
        pid, plid, ac = lax.fori_loop(0, nch, chunk, init)

        @pl.when((plid >= 0) & (plid < NPW))
        def _():
            ob = outb.at[plid]
            for k in range(8):
                ob[pl.ds(k * 16, 16)] = ac[k]

        pltpu.sync_copy(outb, out_h.at[pl.ds(nstart, NPW)])


_SC_SEGMAX = pl.kernel(
    _sc_segmax_body,
    out_type=[jax.ShapeDtypeStruct((NPAD, EMB), jnp.float32),
              jax.ShapeDtypeStruct((NPAD, EMB), jnp.float32)],
    mesh=_MESH,
    scratch_types=[pltpu.VMEM((NPW, EMB), jnp.float32),
                   pltpu.VMEM((SCH, EMB), jnp.float32),
                   pltpu.VMEM((SCH,), jnp.int32),
                   pltpu.VMEM((SCH + 16,), jnp.int32),
                   pltpu.VMEM((24,), jnp.int32),
                   pltpu.SemaphoreType.DMA],
)


def _sorted_index_setup(ids):
    order = jnp.argsort(ids).astype(jnp.int32)
    ids_sorted = ids[order].astype(jnp.int32)
    rs = jnp.searchsorted(ids_sorted, jnp.arange(RSP, dtype=jnp.int32),
                          side="left").astype(jnp.int32)
    ids_pad = jnp.concatenate(
        [ids_sorted, jnp.full((EPAD - E_,), 1 << 22, jnp.int32)])
    perm_pad = jnp.concatenate(
        [order, jnp.zeros((EPAD - E_,), jnp.int32)])
    return perm_pad, ids_pad, rs


def kernel(nodes, edges, globals_, senders, receivers, We_emb, be_emb,
           Wn_emb, bn_emb, Wg_emb, bg_emb, W_edge, b_edge, W_node, b_node,
           W_glob, b_glob, W_dec, b_dec):
    e = _mm(edges, We_emb, be_emb, blk=2000)
    n = _mm(nodes, Wn_emb, bn_emb, blk=2000)
    g = globals_ @ Wg_emb + bg_emb

    perm_s, ids_s, rs_s = _sorted_index_setup(senders)
    perm_r, ids_r, rs_r = _sorted_index_setup(receivers)

    We1, We2, We3, We4 = (W_edge[:, i * EMB:(i + 1) * EMB, :] for i in range(4))
    Wn1, Wn2, Wn3, Wn4 = (W_node[:, i * EMB:(i + 1) * EMB, :] for i in range(4))
    Wg1 = W_glob[:, :EMB]
    Wg2 = W_glob[:, EMB:2 * EMB]
    Wg3 = W_glob[:, 2 * EMB:]

    for l in range(3):
        ec = (g @ We4[l] + b_edge[l]).reshape(1, EMB)
        ts, tr = _mm2(n, We2[l], We3[l], blk=2000)
        gs, gr = _SC_GATHER(ts, tr, senders, receivers)
        e, agge = _edge_up(e, gs, gr, We1[l], ec)

        sent, recv = _SC_SEGMAX(e, perm_s, ids_s, rs_s, perm_r, ids_r, rs_r)

        nc = (g @ Wn4[l] + b_node[l]).reshape(1, EMB)
        n, aggn = _node_up(n, sent, recv, Wn1[l], Wn2[l], Wn3[l], nc)

        g = g + jax.nn.relu(aggn @ Wg1[l] + agge @ Wg2[l] + g @ Wg3[l]
                            + b_glob[l])

    e_dec = _mm(e, W_dec, b_dec, blk=2000)
    return n, e_dec, g


# speedup vs baseline: 1.9243x; 1.1185x over previous
import functools

import jax
import jax.numpy as jnp
from jax import lax
from jax.experimental import pallas as pl
from jax.experimental.pallas import tpu as pltpu
from jax.experimental.pallas import tpu_sc as plsc

EMB = 128
E_ = 160000
N_ = 10000
NC = 2
NS = 16
NW = NC * NS
EPW = E_ // NW
SCH = 128
NCHE = EPW // SCH + 1
NPW = 320
NPAD = NW * NPW
EPAD = E_ + 320
RSP = 10304


def _mm_kernel(x_ref, w_ref, b_ref, o_ref):
    o_ref[...] = (
        jnp.dot(x_ref[...], w_ref[...], preferred_element_type=jnp.float32)
        + b_ref[...]
    )


def _mm(x, w, b, blk):
    M, K = x.shape
    N = w.shape[1]
    return pl.pallas_call(
        _mm_kernel,
        grid=(M // blk,),
        in_specs=[
            pl.BlockSpec((blk, K), lambda i: (i, 0)),
            pl.BlockSpec((K, N), lambda i: (0, 0)),
            pl.BlockSpec((1, N), lambda i: (0, 0)),
        ],
        out_specs=pl.BlockSpec((blk, N), lambda i: (i, 0)),
        out_shape=jax.ShapeDtypeStruct((M, N), jnp.float32),
    )(x, w, b.reshape(1, N))


def _mm_nb_kernel(x_ref, w_ref, o_ref):
    o_ref[...] = jnp.dot(x_ref[...], w_ref[...],
                         preferred_element_type=jnp.float32)


def _mm_nb(x, w, blk):
    M, K = x.shape
    N = w.shape[1]
    return pl.pallas_call(
        _mm_nb_kernel,
        grid=(M // blk,),
        in_specs=[
            pl.BlockSpec((blk, K), lambda i: (i, 0)),
            pl.BlockSpec((K, N), lambda i: (0, 0)),
        ],
        out_specs=pl.BlockSpec((blk, N), lambda i: (i, 0)),
        out_shape=jax.ShapeDtypeStruct((M, N), jnp.float32),
    )(x, w)


def _mm2_kernel(x_ref, w1_ref, w2_ref, o1_ref, o2_ref):
    o1_ref[...] = jnp.dot(x_ref[...], w1_ref[...],
                          preferred_element_type=jnp.float32)
    o2_ref[...] = jnp.dot(x_ref[...], w2_ref[...],
                          preferred_element_type=jnp.float32)


def _mm2(x, w1, w2, blk):
    M, K = x.shape
    N = w1.shape[1]
    dspec = pl.BlockSpec((blk, N), lambda i: (i, 0))
    return pl.pallas_call(
        _mm2_kernel,
        grid=(M // blk,),
        in_specs=[
            pl.BlockSpec((blk, K), lambda i: (i, 0)),
            pl.BlockSpec((K, N), lambda i: (0, 0)),
            pl.BlockSpec((K, N), lambda i: (0, 0)),
        ],
        out_specs=[dspec, dspec],
        out_shape=[jax.ShapeDtypeStruct((M, N), jnp.float32)] * 2,
    )(x, w1, w2)


def _node_up_kernel(n_ref, s_ref, r_ref, w1_ref, w2_ref, w3_ref, c_ref,
                    o_ref, aggn_ref):
    acc = jnp.dot(n_ref[...], w1_ref[...], preferred_element_type=jnp.float32)
    acc += jnp.dot(s_ref[...], w2_ref[...], preferred_element_type=jnp.float32)
    acc += jnp.dot(r_ref[...], w3_ref[...], preferred_element_type=jnp.float32)
    acc += c_ref[...]
    nn = n_ref[...] + jnp.maximum(acc, 0.0)
    o_ref[...] = nn
    m = jnp.max(nn, axis=0, keepdims=True)
    i = pl.program_id(0)

    @pl.when(i == 0)
    def _():
        aggn_ref[...] = m

    @pl.when(i > 0)
    def _():
        aggn_ref[...] = jnp.maximum(aggn_ref[...], m)


def _node_up(n, s, r, w1, w2, w3, c, blk=2000):
    M = n.shape[0]
    dspec = pl.BlockSpec((blk, EMB), lambda i: (i, 0))
    wspec = pl.BlockSpec((EMB, EMB), lambda i: (0, 0))
    cspec = pl.BlockSpec((1, EMB), lambda i: (0, 0))
    return pl.pallas_call(
        _node_up_kernel,
        grid=(M // blk,),
        in_specs=[dspec, dspec, dspec, wspec, wspec, wspec, cspec],
        out_specs=[dspec, cspec],
        out_shape=[jax.ShapeDtypeStruct((M, EMB), jnp.float32),
                   jax.ShapeDtypeStruct((1, EMB), jnp.float32)],
    )(n, s, r, w1, w2, w3, c)


_MESH = plsc.VectorSubcoreMesh(core_axis_name="c", subcore_axis_name="s")


def _sc_gather_body(ts_hbm, tr_hbm, snd_hbm, rcv_hbm,
                    gs_hbm, gr_hbm,
                    tsb0, tsb1, trb0, trb1, si0, si1, ri0, ri1, sem):
    wid = lax.axis_index("s") * NC + lax.axis_index("c")
    base = wid * EPW
    bufs = ((tsb0, trb0, si0, ri0), (tsb1, trb1, si1, ri1))

    def off_of(c):
        return base + pl.multiple_of(
            jnp.where(c == NCHE - 1, EPW - SCH, c * SCH), 8)

    def fire(c, b):
        tsb, trb, si, ri = bufs[b]
        off = off_of(c)
        pltpu.sync_copy(snd_hbm.at[pl.ds(off, SCH)], si)
        pltpu.sync_copy(rcv_hbm.at[pl.ds(off, SCH)], ri)
        pltpu.async_copy(ts_hbm.at[si], tsb, sem)
        pltpu.async_copy(tr_hbm.at[ri], trb, sem)

    fire(0, 0)

    def group(g, _):
        for b in range(2):
            c = 2 * g + b
            tsb, trb, si, ri = bufs[b]
            pltpu.make_async_copy(ts_hbm.at[si], tsb, sem).wait()
            pltpu.make_async_copy(tr_hbm.at[ri], trb, sem).wait()

            @pl.when(c + 1 < NCHE)
            def _(c=c, b=b):
                fire(c + 1, 1 - b)

            off = off_of(c)
            pltpu.sync_copy(tsb, gs_hbm.at[pl.ds(off, SCH)])
            pltpu.sync_copy(trb, gr_hbm.at[pl.ds(off, SCH)])
        return 0

    lax.fori_loop(0, NCHE // 2, group, 0)


_SC_GATHER = pl.kernel(
    _sc_gather_body,
    out_type=[jax.ShapeDtypeStruct((E_, EMB), jnp.float32),
              jax.ShapeDtypeStruct((E_, EMB), jnp.float32)],
    mesh=_MESH,
    scratch_types=[pltpu.VMEM((SCH, EMB), jnp.float32),
                   pltpu.VMEM((SCH, EMB), jnp.float32),
                   pltpu.VMEM((SCH, EMB), jnp.float32),
                   pltpu.VMEM((SCH, EMB), jnp.float32),
                   pltpu.VMEM((SCH,), jnp.int32),
                   pltpu.VMEM((SCH,), jnp.int32),
                   pltpu.VMEM((SCH,), jnp.int32),
                   pltpu.VMEM((SCH,), jnp.int32),
                   pltpu.SemaphoreType.DMA],
)


def _edge_up_kernel(e_ref, gs_ref, gr_ref, w_ref, c_ref, o_ref, agg_ref):
    acc = jnp.dot(e_ref[...], w_ref[...], preferred_element_type=jnp.float32)
    acc += gs_ref[...] + gr_ref[...] + c_ref[...]
    nn = e_ref[...] + jnp.maximum(acc, 0.0)
    o_ref[...] = nn
    m = jnp.max(nn, axis=0, keepdims=True)
    i = pl.program_id(0)

    @pl.when(i == 0)
    def _():
        agg_ref[...] = m

    @pl.when(i > 0)
    def _():
        agg_ref[...] = jnp.maximum(agg_ref[...], m)


def _edge_up(e, gs, gr, w, c, blk=2000):
    M = e.shape[0]
    dspec = pl.BlockSpec((blk, EMB), lambda i: (i, 0))
    wspec = pl.BlockSpec((EMB, EMB), lambda i: (0, 0))
    cspec = pl.BlockSpec((1, EMB), lambda i: (0, 0))
    return pl.pallas_call(
        _edge_up_kernel,
        grid=(M // blk,),
        in_specs=[dspec, dspec, dspec, wspec, cspec],
        out_specs=[dspec, cspec],
        out_shape=[jax.ShapeDtypeStruct((M, EMB), jnp.float32),
                   jax.ShapeDtypeStruct((1, EMB), jnp.float32)],
    )(e, gs, gr, w, c)


def _sc_segmax_body(enew_hbm, perm_s_hbm, ids_s_hbm, rs_s_hbm,
                    perm_r_hbm, ids_r_hbm, rs_r_hbm,
                    sent_hbm, recv_hbm,
                    outb, rows0, rows1, pv0, pv1, iv0, iv1, rsv, sem):
    wid = lax.axis_index("s") * NC + lax.axis_index("c")
    nstart = wid * NPW
    zero16 = jnp.zeros((16,), jnp.float32)
    bufs = ((iv0, pv0, rows0), (iv1, pv1, rows1))

    def read_rs(rs_h, pos):
        a = pl.multiple_of((pos // 8) * 8, 8)
        pltpu.sync_copy(rs_h.at[pl.ds(a, 24)], rsv)
        return rsv[pl.ds(pos - a, 16)][0]

    for perm_h, ids_h, rs_h, out_h in (
            (perm_s_hbm, ids_s_hbm, rs_s_hbm, sent_hbm),
            (perm_r_hbm, ids_r_hbm, rs_r_hbm, recv_hbm)):

        def zr(t, _):
            ob = outb.at[t]
            for k in range(8):
                ob[pl.ds(k * 16, 16)] = zero16
            return 0

        lax.fori_loop(0, NPW, zr, 0)

        est = read_rs(rs_h, nstart)
        een = read_rs(rs_h, nstart + NPW)
        est_a = pl.multiple_of((est // 8) * 8, 8)
        nch = (een - est_a + SCH - 1) // SCH
        nch2 = ((nch + 1) // 2) * 2

        def fire(c, b, perm_h=perm_h, ids_h=ids_h):
            iv, pv, rows = bufs[b]
            off = pl.multiple_of(est_a + c * SCH, 8)
            pltpu.sync_copy(ids_h.at[pl.ds(off, SCH + 16)], iv)
            pltpu.sync_copy(perm_h.at[pl.ds(off, SCH)], pv)
            pltpu.async_copy(enew_hbm.at[pv], rows, sem)

        @pl.when(nch2 > 0)
        def _(fire=fire):
            fire(0, 0)

        def group(g, carry, fire=fire):
            for b in range(2):
                c = 2 * g + b
                iv, pv, rows = bufs[b]
                pltpu.make_async_copy(enew_hbm.at[pv], rows, sem).wait()

                @pl.when(c + 1 < nch2)
                def _(c=c, b=b, fire=fire):
                    fire(c + 1, 1 - b)

                def row(j, cr, iv=iv, rows=rows):
                    pid, plid, ac = cr
                    idd = iv[pl.ds(j, 16)][0]
                    lid = idd - nstart
                    is_new = idd != pid

                    @pl.when(is_new & (plid >= 0) & (plid < NPW))
                    def _():
                        ob = outb.at[plid]
                        for k in range(8):
                            ob[pl.ds(k * 16, 16)] = ac[k]

                    rj = rows.at[j]
                    nac = []
                    for k in range(8):
                        rv = rj[pl.ds(k * 16, 16)]
                        nac.append(jnp.where(is_new, rv,
                                             jnp.maximum(ac[k], rv)))
                    return (idd, lid, tuple(nac))

                carry = lax.fori_loop(0, SCH, row, carry)
            return carry

        init = (jnp.int32(-(1 << 30)), jnp.int32(-1),
                tuple(zero16 for _ in range(8)))
        pid, plid, ac = lax.fori_loop(0, nch2 // 2, group, init)

        @pl.when((plid >= 0) & (plid < NPW))
        def _():
            ob = outb.at[plid]
            for k in range(8):
                ob[pl.ds(k * 16, 16)] = ac[k]

        pltpu.sync_copy(outb, out_h.at[pl.ds(nstart, NPW)])


_SC_SEGMAX = pl.kernel(
    _sc_segmax_body,
    out_type=[jax.ShapeDtypeStruct((NPAD, EMB), jnp.float32),
              jax.ShapeDtypeStruct((NPAD, EMB), jnp.float32)],
    mesh=_MESH,
    scratch_types=[pltpu.VMEM((NPW, EMB), jnp.float32),
                   pltpu.VMEM((SCH, EMB), jnp.float32),
                   pltpu.VMEM((SCH, EMB), jnp.float32),
                   pltpu.VMEM((SCH,), jnp.int32),
                   pltpu.VMEM((SCH,), jnp.int32),
                   pltpu.VMEM((SCH + 16,), jnp.int32),
                   pltpu.VMEM((SCH + 16,), jnp.int32),
                   pltpu.VMEM((24,), jnp.int32),
                   pltpu.SemaphoreType.DMA],
)


def _sorted_index_setup(ids):
    order = jnp.argsort(ids).astype(jnp.int32)
    ids_sorted = ids[order].astype(jnp.int32)
    rs = jnp.searchsorted(ids_sorted, jnp.arange(RSP, dtype=jnp.int32),
                          side="left").astype(jnp.int32)
    ids_pad = jnp.concatenate(
        [ids_sorted, jnp.full((EPAD - E_,), 1 << 22, jnp.int32)])
    perm_pad = jnp.concatenate(
        [order, jnp.zeros((EPAD - E_,), jnp.int32)])
    return perm_pad, ids_pad, rs


def kernel(nodes, edges, globals_, senders, receivers, We_emb, be_emb,
           Wn_emb, bn_emb, Wg_emb, bg_emb, W_edge, b_edge, W_node, b_node,
           W_glob, b_glob, W_dec, b_dec):
    e = _mm(edges, We_emb, be_emb, blk=2000)
    n = _mm(nodes, Wn_emb, bn_emb, blk=2000)
    g = globals_ @ Wg_emb + bg_emb

    perm_s, ids_s, rs_s = _sorted_index_setup(senders)
    perm_r, ids_r, rs_r = _sorted_index_setup(receivers)

    We1, We2, We3, We4 = (W_edge[:, i * EMB:(i + 1) * EMB, :] for i in range(4))
    Wn1, Wn2, Wn3, Wn4 = (W_node[:, i * EMB:(i + 1) * EMB, :] for i in range(4))
    Wg1 = W_glob[:, :EMB]
    Wg2 = W_glob[:, EMB:2 * EMB]
    Wg3 = W_glob[:, 2 * EMB:]

    for l in range(3):
        ec = (g @ We4[l] + b_edge[l]).reshape(1, EMB)
        ts, tr = _mm2(n, We2[l], We3[l], blk=2000)
        gs, gr = _SC_GATHER(ts, tr, senders, receivers)
        e, agge = _edge_up(e, gs, gr, We1[l], ec)

        sent, recv = _SC_SEGMAX(e, perm_s, ids_s, rs_s, perm_r, ids_r, rs_r)

        nc = (g @ Wn4[l] + b_node[l]).reshape(1, EMB)
        n, aggn = _node_up(n, sent, recv, Wn1[l], Wn2[l], Wn3[l], nc)

        g = g + jax.nn.relu(aggn @ Wg1[l] + agge @ Wg2[l] + g @ Wg3[l]
                            + b_glob[l])

    e_dec = _mm(e, W_dec, b_dec, blk=2000)
    return n, e_dec, g


# speedup vs baseline: 1.9903x; 1.0343x over previous
import functools

import jax
import jax.numpy as jnp
from jax import lax
from jax.experimental import pallas as pl
from jax.experimental.pallas import tpu as pltpu
from jax.experimental.pallas import tpu_sc as plsc

EMB = 128
E_ = 160000
N_ = 10000
NC = 2
NS = 16
NW = NC * NS
EPW = E_ // NW
SCH = 128
NCHE = EPW // SCH + 1
NPW = 320
NPAD = NW * NPW
EPAD = E_ + 320
RSP = 10304


def _mm_kernel(x_ref, w_ref, b_ref, o_ref):
    o_ref[...] = (
        jnp.dot(x_ref[...], w_ref[...], preferred_element_type=jnp.float32)
        + b_ref[...]
    )


def _mm(x, w, b, blk):
    M, K = x.shape
    N = w.shape[1]
    return pl.pallas_call(
        _mm_kernel,
        grid=(M // blk,),
        in_specs=[
            pl.BlockSpec((blk, K), lambda i: (i, 0)),
            pl.BlockSpec((K, N), lambda i: (0, 0)),
            pl.BlockSpec((1, N), lambda i: (0, 0)),
        ],
        out_specs=pl.BlockSpec((blk, N), lambda i: (i, 0)),
        out_shape=jax.ShapeDtypeStruct((M, N), jnp.float32),
    )(x, w, b.reshape(1, N))


def _mm_nb_kernel(x_ref, w_ref, o_ref):
    o_ref[...] = jnp.dot(x_ref[...], w_ref[...],
                         preferred_element_type=jnp.float32)


def _mm_nb(x, w, blk):
    M, K = x.shape
    N = w.shape[1]
    return pl.pallas_call(
        _mm_nb_kernel,
        grid=(M // blk,),
        in_specs=[
            pl.BlockSpec((blk, K), lambda i: (i, 0)),
            pl.BlockSpec((K, N), lambda i: (0, 0)),
        ],
        out_specs=pl.BlockSpec((blk, N), lambda i: (i, 0)),
        out_shape=jax.ShapeDtypeStruct((M, N), jnp.float32),
    )(x, w)


def _mm2_kernel(x_ref, w1_ref, w2_ref, o1_ref, o2_ref):
    o1_ref[...] = jnp.dot(x_ref[...], w1_ref[...],
                          preferred_element_type=jnp.float32)
    o2_ref[...] = jnp.dot(x_ref[...], w2_ref[...],
                          preferred_element_type=jnp.float32)


def _mm2(x, w1, w2, blk):
    M, K = x.shape
    N = w1.shape[1]
    dspec = pl.BlockSpec((blk, N), lambda i: (i, 0))
    return pl.pallas_call(
        _mm2_kernel,
        grid=(M // blk,),
        in_specs=[
            pl.BlockSpec((blk, K), lambda i: (i, 0)),
            pl.BlockSpec((K, N), lambda i: (0, 0)),
            pl.BlockSpec((K, N), lambda i: (0, 0)),
        ],
        out_specs=[dspec, dspec],
        out_shape=[jax.ShapeDtypeStruct((M, N), jnp.float32)] * 2,
    )(x, w1, w2)


def _node_up_kernel(n_ref, s_ref, r_ref, w1_ref, w2_ref, w3_ref, c_ref,
                    o_ref, aggn_ref):
    acc = jnp.dot(n_ref[...], w1_ref[...], preferred_element_type=jnp.float32)
    acc += jnp.dot(s_ref[...], w2_ref[...], preferred_element_type=jnp.float32)
    acc += jnp.dot(r_ref[...], w3_ref[...], preferred_element_type=jnp.float32)
    acc += c_ref[...]
    nn = n_ref[...] + jnp.maximum(acc, 0.0)
    o_ref[...] = nn
    m = jnp.max(nn, axis=0, keepdims=True)
    i = pl.program_id(0)

    @pl.when(i == 0)
    def _():
        aggn_ref[...] = m

    @pl.when(i > 0)
    def _():
        aggn_ref[...] = jnp.maximum(aggn_ref[...], m)


def _node_up(n, s, r, w1, w2, w3, c, blk=2000):
    M = n.shape[0]
    dspec = pl.BlockSpec((blk, EMB), lambda i: (i, 0))
    wspec = pl.BlockSpec((EMB, EMB), lambda i: (0, 0))
    cspec = pl.BlockSpec((1, EMB), lambda i: (0, 0))
    return pl.pallas_call(
        _node_up_kernel,
        grid=(M // blk,),
        in_specs=[dspec, dspec, dspec, wspec, wspec, wspec, cspec],
        out_specs=[dspec, cspec],
        out_shape=[jax.ShapeDtypeStruct((M, EMB), jnp.float32),
                   jax.ShapeDtypeStruct((1, EMB), jnp.float32)],
    )(n, s, r, w1, w2, w3, c)


_MESH = plsc.VectorSubcoreMesh(core_axis_name="c", subcore_axis_name="s")


def _sc_gather_body(ts_hbm, tr_hbm, snd_hbm, rcv_hbm,
                    gsum_hbm,
                    tsb0, tsb1, trb0, trb1, si0, si1, ri0, ri1, sem):
    wid = lax.axis_index("s") * NC + lax.axis_index("c")
    base = wid * EPW
    bufs = ((tsb0, trb0, si0, ri0), (tsb1, trb1, si1, ri1))

    def off_of(c):
        return base + pl.multiple_of(
            jnp.where(c == NCHE - 1, EPW - SCH, c * SCH), 8)

    def fire(c, b):
        tsb, trb, si, ri = bufs[b]
        off = off_of(c)
        pltpu.sync_copy(snd_hbm.at[pl.ds(off, SCH)], si)
        pltpu.sync_copy(rcv_hbm.at[pl.ds(off, SCH)], ri)
        pltpu.async_copy(ts_hbm.at[si], tsb, sem)
        pltpu.async_copy(tr_hbm.at[ri], trb, sem)

    fire(0, 0)

    def group(g, _):
        for b in range(2):
            c = 2 * g + b
            tsb, trb, si, ri = bufs[b]
            pltpu.make_async_copy(ts_hbm.at[si], tsb, sem).wait()
            pltpu.make_async_copy(tr_hbm.at[ri], trb, sem).wait()

            @pl.when(c + 1 < NCHE)
            def _(c=c, b=b):
                fire(c + 1, 1 - b)

            def add_row(t, _, tsb=tsb, trb=trb):
                a = tsb.at[t]
                bb = trb.at[t]
                for k in range(8):
                    a[pl.ds(k * 16, 16)] = (a[pl.ds(k * 16, 16)]
                                            + bb[pl.ds(k * 16, 16)])
                return 0

            lax.fori_loop(0, SCH, add_row, 0)
            off = off_of(c)
            pltpu.sync_copy(tsb, gsum_hbm.at[pl.ds(off, SCH)])
        return 0

    lax.fori_loop(0, NCHE // 2, group, 0)


_SC_GATHER = pl.kernel(
    _sc_gather_body,
    out_type=jax.ShapeDtypeStruct((E_, EMB), jnp.float32),
    mesh=_MESH,
    scratch_types=[pltpu.VMEM((SCH, EMB), jnp.float32),
                   pltpu.VMEM((SCH, EMB), jnp.float32),
                   pltpu.VMEM((SCH, EMB), jnp.float32),
                   pltpu.VMEM((SCH, EMB), jnp.float32),
                   pltpu.VMEM((SCH,), jnp.int32),
                   pltpu.VMEM((SCH,), jnp.int32),
                   pltpu.VMEM((SCH,), jnp.int32),
                   pltpu.VMEM((SCH,), jnp.int32),
                   pltpu.SemaphoreType.DMA],
)


def _edge_up_kernel(e_ref, gsum_ref, w_ref, c_ref, o_ref, agg_ref):
    acc = jnp.dot(e_ref[...], w_ref[...], preferred_element_type=jnp.float32)
    acc += gsum_ref[...] + c_ref[...]
    nn = e_ref[...] + jnp.maximum(acc, 0.0)
    o_ref[...] = nn
    m = jnp.max(nn, axis=0, keepdims=True)
    i = pl.program_id(0)

    @pl.when(i == 0)
    def _():
        agg_ref[...] = m

    @pl.when(i > 0)
    def _():
        agg_ref[...] = jnp.maximum(agg_ref[...], m)


def _edge_up(e, gsum, w, c, blk=2000):
    M = e.shape[0]
    dspec = pl.BlockSpec((blk, EMB), lambda i: (i, 0))
    wspec = pl.BlockSpec((EMB, EMB), lambda i: (0, 0))
    cspec = pl.BlockSpec((1, EMB), lambda i: (0, 0))
    return pl.pallas_call(
        _edge_up_kernel,
        grid=(M // blk,),
        in_specs=[dspec, dspec, wspec, cspec],
        out_specs=[dspec, cspec],
        out_shape=[jax.ShapeDtypeStruct((M, EMB), jnp.float32),
                   jax.ShapeDtypeStruct((1, EMB), jnp.float32)],
    )(e, gsum, w, c)


def _sc_segmax_body(enew_hbm, perm_s_hbm, ids_s_hbm, rs_s_hbm,
                    perm_r_hbm, ids_r_hbm, rs_r_hbm,
                    sent_hbm, recv_hbm,
                    outb, rows0, rows1, pv0, pv1, iv0, iv1, rsv, sem):
    wid = lax.axis_index("s") * NC + lax.axis_index("c")
    nstart = wid * NPW
    zero16 = jnp.zeros((16,), jnp.float32)
    bufs = ((iv0, pv0, rows0), (iv1, pv1, rows1))

    def read_rs(rs_h, pos):
        a = pl.multiple_of((pos // 8) * 8, 8)
        pltpu.sync_copy(rs_h.at[pl.ds(a, 24)], rsv)
        return rsv[pl.ds(pos - a, 16)][0]

    for perm_h, ids_h, rs_h, out_h in (
            (perm_s_hbm, ids_s_hbm, rs_s_hbm, sent_hbm),
            (perm_r_hbm, ids_r_hbm, rs_r_hbm, recv_hbm)):

        def zr(t, _):
            ob = outb.at[t]
            for k in range(8):
                ob[pl.ds(k * 16, 16)] = zero16
            return 0

        lax.fori_loop(0, NPW, zr, 0)

        est = read_rs(rs_h, nstart)
        een = read_rs(rs_h, nstart + NPW)
        est_a = pl.multiple_of((est // 8) * 8, 8)
        nch = (een - est_a + SCH - 1) // SCH
        nch2 = ((nch + 1) // 2) * 2

        def fire(c, b, perm_h=perm_h, ids_h=ids_h):
            iv, pv, rows = bufs[b]
            off = pl.multiple_of(est_a + c * SCH, 8)
            pltpu.sync_copy(ids_h.at[pl.ds(off, SCH + 16)], iv)
            pltpu.sync_copy(perm_h.at[pl.ds(off, SCH)], pv)
            pltpu.async_copy(enew_hbm.at[pv], rows, sem)

        @pl.when(nch2 > 0)
        def _(fire=fire):
            fire(0, 0)

        def group(g, carry, fire=fire):
            for b in range(2):
                c = 2 * g + b
                iv, pv, rows = bufs[b]
                pltpu.make_async_copy(enew_hbm.at[pv], rows, sem).wait()

                @pl.when(c + 1 < nch2)
                def _(c=c, b=b, fire=fire):
                    fire(c + 1, 1 - b)

                def row(j, cr, iv=iv, rows=rows):
                    pid, plid, ac = cr
                    idd = iv[pl.ds(j, 16)][0]
                    lid = idd - nstart
                    is_new = idd != pid

                    @pl.when(is_new & (plid >= 0) & (plid < NPW))
                    def _():
                        ob = outb.at[plid]
                        for k in range(8):
                            ob[pl.ds(k * 16, 16)] = ac[k]

                    rj = rows.at[j]
                    nac = []
                    for k in range(8):
                        rv = rj[pl.ds(k * 16, 16)]
                        nac.append(jnp.where(is_new, rv,
                                             jnp.maximum(ac[k], rv)))
                    return (idd, lid, tuple(nac))

                carry = lax.fori_loop(0, SCH, row, carry)
            return carry

        init = (jnp.int32(-(1 << 30)), jnp.int32(-1),
                tuple(zero16 for _ in range(8)))
        pid, plid, ac = lax.fori_loop(0, nch2 // 2, group, init)

        @pl.when((plid >= 0) & (plid < NPW))
        def _():
            ob = outb.at[plid]
            for k in range(8):
                ob[pl.ds(k * 16, 16)] = ac[k]

        pltpu.sync_copy(outb, out_h.at[pl.ds(nstart, NPW)])


_SC_SEGMAX = pl.kernel(
    _sc_segmax_body,
    out_type=[jax.ShapeDtypeStruct((NPAD, EMB), jnp.float32),
              jax.ShapeDtypeStruct((NPAD, EMB), jnp.float32)],
    mesh=_MESH,
    scratch_types=[pltpu.VMEM((NPW, EMB), jnp.float32),
                   pltpu.VMEM((SCH, EMB), jnp.float32),
                   pltpu.VMEM((SCH, EMB), jnp.float32),
                   pltpu.VMEM((SCH,), jnp.int32),
                   pltpu.VMEM((SCH,), jnp.int32),
                   pltpu.VMEM((SCH + 16,), jnp.int32),
                   pltpu.VMEM((SCH + 16,), jnp.int32),
                   pltpu.VMEM((24,), jnp.int32),
                   pltpu.SemaphoreType.DMA],
)


def _sorted_index_setup(ids):
    order = jnp.argsort(ids).astype(jnp.int32)
    ids_sorted = ids[order].astype(jnp.int32)
    rs = jnp.searchsorted(ids_sorted, jnp.arange(RSP, dtype=jnp.int32),
                          side="left").astype(jnp.int32)
    ids_pad = jnp.concatenate(
        [ids_sorted, jnp.full((EPAD - E_,), 1 << 22, jnp.int32)])
    perm_pad = jnp.concatenate(
        [order, jnp.zeros((EPAD - E_,), jnp.int32)])
    return perm_pad, ids_pad, rs


def kernel(nodes, edges, globals_, senders, receivers, We_emb, be_emb,
           Wn_emb, bn_emb, Wg_emb, bg_emb, W_edge, b_edge, W_node, b_node,
           W_glob, b_glob, W_dec, b_dec):
    e = _mm(edges, We_emb, be_emb, blk=2000)
    n = _mm(nodes, Wn_emb, bn_emb, blk=2000)
    g = globals_ @ Wg_emb + bg_emb

    perm_s, ids_s, rs_s = _sorted_index_setup(senders)
    perm_r, ids_r, rs_r = _sorted_index_setup(receivers)

    We1, We2, We3, We4 = (W_edge[:, i * EMB:(i + 1) * EMB, :] for i in range(4))
    Wn1, Wn2, Wn3, Wn4 = (W_node[:, i * EMB:(i + 1) * EMB, :] for i in range(4))
    Wg1 = W_glob[:, :EMB]
    Wg2 = W_glob[:, EMB:2 * EMB]
    Wg3 = W_glob[:, 2 * EMB:]

    for l in range(3):
        ec = (g @ We4[l] + b_edge[l]).reshape(1, EMB)
        ts, tr = _mm2(n, We2[l], We3[l], blk=2000)
        gsum = _SC_GATHER(ts, tr, senders, receivers)
        e, agge = _edge_up(e, gsum, We1[l], ec)

        sent, recv = _SC_SEGMAX(e, perm_s, ids_s, rs_s, perm_r, ids_r, rs_r)

        nc = (g @ Wn4[l] + b_node[l]).reshape(1, EMB)
        n, aggn = _node_up(n, sent, recv, Wn1[l], Wn2[l], Wn3[l], nc)

        g = g + jax.nn.relu(aggn @ Wg1[l] + agge @ Wg2[l] + g @ Wg3[l]
                            + b_glob[l])

    e_dec = _mm(e, W_dec, b_dec, blk=2000)
    return n, e_dec, g
